# baseline (speedup 1.0000x reference)
import functools

import jax
import jax.numpy as jnp
from jax import lax
from jax.experimental import pallas as pl
from jax.experimental.pallas import tpu as pltpu
from jax.experimental.pallas import tpu_sc as plsc

_N = 10000
_NP = 10240
_E = 160000
_D = 128
_DT = 64
_NI = 8
_NET = 9
_NNT = 3
_LAYERS = 3

_NW = 32
_C = 128
_EPAD = 163840
_EPT = _EPAD // _NW
_NCH = _EPT // _C
_RPS = _NP // 16

_BN = 512
_SELU_SCALE = 1.0507009873554805
_SELU_ALPHA = 1.6732632423543772


def _softmax(z):
    m = jnp.max(z, axis=-1, keepdims=True)
    e = jnp.exp(z - m)
    return e / jnp.sum(e, axis=-1, keepdims=True)


def _tc_body(x_ref, wmsg_ref, wsrc_ref, bsrc_ref, etv_ref, wint_ref, bint_ref,
             wibn_ref, bibn_ref, a_ref, v_ref, u_ref):
    xb = x_ref[...]
    for t in range(_NET):
        a_ref[t] = jnp.dot(xb, wmsg_ref[t], preferred_element_type=jnp.float32)
        h = jnp.dot(xb, wsrc_ref[t], preferred_element_type=jnp.float32) + bsrc_ref[t][None, :]
        s = _SELU_SCALE * jnp.where(h > 0, h, _SELU_ALPHA * (jnp.exp(h) - 1.0))
        v_ref[t] = jnp.sum(s * etv_ref[t][None, :], axis=1)
    for t in range(_NNT):
        ti = _softmax(jnp.dot(xb, wint_ref[t], preferred_element_type=jnp.float32) + bint_ref[t][None, :])
        tw = _softmax(jnp.dot(ti, wibn_ref[t], preferred_element_type=jnp.float32) + bibn_ref[t][None, :])
        u_ref[t] = jnp.concatenate(
            [tw, jnp.zeros((tw.shape[0], 128 - _NET), jnp.float32)], axis=1)


_tc_pre = pl.pallas_call(
    _tc_body,
    grid=(_NP // _BN,),
    in_specs=[
        pl.BlockSpec((_BN, _D), lambda i: (i, 0)),
        pl.BlockSpec((_NET, _D, _D), lambda i: (0, 0, 0)),
        pl.BlockSpec((_NET, _D, _DT), lambda i: (0, 0, 0)),
        pl.BlockSpec((_NET, _DT), lambda i: (0, 0)),
        pl.BlockSpec((_NET, _DT), lambda i: (0, 0)),
        pl.BlockSpec((_NNT, _D, _NI), lambda i: (0, 0, 0)),
        pl.BlockSpec((_NNT, _NI), lambda i: (0, 0)),
        pl.BlockSpec((_NNT, _NI, _NET), lambda i: (0, 0, 0)),
        pl.BlockSpec((_NNT, _NET), lambda i: (0, 0)),
    ],
    out_specs=[
        pl.BlockSpec((_NET, _BN, _D), lambda i: (0, i, 0)),
        pl.BlockSpec((_NET, _BN), lambda i: (0, i)),
        pl.BlockSpec((_NNT, _BN, 128), lambda i: (0, i, 0)),
    ],
    out_shape=[
        jax.ShapeDtypeStruct((_NET, _NP, _D), jnp.float32),
        jax.ShapeDtypeStruct((_NET, _NP), jnp.float32),
        jax.ShapeDtypeStruct((_NNT, _NP, 128), jnp.float32),
    ],
)

_sc_mesh = plsc.VectorSubcoreMesh(core_axis_name="c", subcore_axis_name="s")
_sc_params = pltpu.CompilerParams(needs_layout_passes=False)


_USZ = _NNT * _NP * _NET
_UPT = _USZ // 16


@functools.partial(
    pl.kernel,
    mesh=_sc_mesh,
    compiler_params=_sc_params,
    out_type=[
        jax.ShapeDtypeStruct((_NW, _NCH, _C), jnp.float32),
        jax.ShapeDtypeStruct((2, _NP), jnp.float32),
    ],
    scratch_types=[
        pltpu.VMEM((_NET * _NP,), jnp.float32),
        pltpu.VMEM((_NCH, _C), jnp.int32),
        pltpu.VMEM((_NCH, _C), jnp.int32),
        pltpu.VMEM((_NCH, _C), jnp.int32),
        pltpu.VMEM((_C,), jnp.float32),
        pltpu.VMEM((_C,), jnp.float32),
        pltpu.VMEM((_NCH, _C), jnp.float32),
        pltpu.VMEM((_RPS,), jnp.float32),
        pltpu.VMEM_SHARED((_NP,), jnp.float32),
        pltpu.SemaphoreType.DMA,
        pltpu.SemaphoreType.DMA,
    ],
)
def _sc_scores(idxv_hbm, tgt_hbm, idxu_hbm, vflat_hbm, u9_hbm,
               es_hbm, denom_hbm,
               vflat_v, idxv_v, tgt_v, idxu_v, ubuf0_v, ubuf1_v, es_v,
               zero_v, denom_sh, g0, g1):
    c = lax.axis_index("c")
    s = lax.axis_index("s")
    wid = c * 16 + s

    def zloop(i, _):
        zero_v[pl.ds(i * 16, 16)] = jnp.zeros((16,), jnp.float32)
        return 0
    lax.fori_loop(0, _RPS // 16, zloop, 0)
    pltpu.sync_copy(zero_v, denom_sh.at[pl.ds(s * _RPS, _RPS)])

    pltpu.sync_copy(vflat_hbm, vflat_v)
    pltpu.sync_copy(idxv_hbm.at[wid], idxv_v)
    pltpu.sync_copy(tgt_hbm.at[wid], tgt_v)
    pltpu.sync_copy(idxu_hbm.at[wid], idxu_v)
    plsc.subcore_barrier()

    pltpu.async_copy(u9_hbm.at[idxu_v.at[0]], ubuf0_v, g0)
    pltpu.async_copy(u9_hbm.at[idxu_v.at[1]], ubuf1_v, g1)

    def finish(ci, ubuf, gsem):
        pltpu.make_async_copy(u9_hbm.at[idxu_v.at[ci]], ubuf, gsem).wait()
        for g in range(_C // 16):
            sl = pl.ds(g * 16, 16)
            v = plsc.load_gather(vflat_v, [idxv_v[ci, sl]])
            u = ubuf[sl]
            gid = (wid * _EPT + ci * _C + g * 16) + lax.iota(jnp.int32, 16)
            es_v[ci, sl] = jnp.where(gid < _E, jnp.exp(v * u), 0.0)
        pltpu.sync_copy(es_v.at[ci], denom_sh.at[tgt_v.at[ci]], add=True)

    def pair(p, _):
        finish(2 * p, ubuf0_v, g0)

        @pl.when(p < _NCH // 2 - 1)
        def _():
            pltpu.async_copy(u9_hbm.at[idxu_v.at[2 * p + 2]], ubuf0_v, g0)
        finish(2 * p + 1, ubuf1_v, g1)

        @pl.when(p < _NCH // 2 - 1)
        def _():
            pltpu.async_copy(u9_hbm.at[idxu_v.at[2 * p + 3]], ubuf1_v, g1)
        return 0
    lax.fori_loop(0, _NCH // 2, pair, 0)

    pltpu.sync_copy(es_v, es_hbm.at[wid])
    plsc.subcore_barrier()
    pltpu.sync_copy(denom_sh.at[pl.ds(s * _RPS, _RPS)],
                    denom_hbm.at[c, pl.ds(s * _RPS, _RPS)])


_HNP = _NP // 2
_EPT2 = _EPAD // 16
_NCH2 = _EPT2 // _C
_RPT2 = _HNP // 16


_C2 = 2 * _C
_PCH = 16


@functools.partial(
    pl.kernel,
    mesh=_sc_mesh,
    compiler_params=_sc_params,
    out_type=jax.ShapeDtypeStruct((_NP, _D), jnp.float32),
    scratch_types=[
        pltpu.VMEM((_NP,), jnp.float32),
        pltpu.VMEM((_PCH, _C), jnp.int32),
        pltpu.VMEM((_PCH, _C), jnp.int32),
        pltpu.VMEM((_PCH, _C), jnp.float32),
        pltpu.VMEM((_EPT2 + _C2,), jnp.int32),
        pltpu.VMEM((_EPT2 + _C2,), jnp.float32),
        pltpu.VMEM((_EPT2 + _C2,), jnp.int32),
        pltpu.VMEM((_C,), jnp.int32),
        pltpu.VMEM((_C,), jnp.int32),
        pltpu.VMEM((_C2, _D), jnp.float32),
        pltpu.VMEM_SHARED((_HNP + 8, _D), jnp.float32),
        pltpu.SemaphoreType.DMA,
    ],
)
def _sc_messages(idxa_hbm, tgt_hbm, es_hbm, denom_hbm, aflat_hbm,
                 out_hbm,
                 inv_v, idxa_v, tgt_v, ese_v, cidx_v, cdist_v,
                 crow_v, rowa_v, rowb_v, abuf_v, out_sh, sem):
    c = lax.axis_index("c")
    s = lax.axis_index("s")
    ab0 = abuf_v

    def zrow(i, _):
        for h in range(_D // 16):
            ab0[i, pl.ds(h * 16, 16)] = jnp.zeros((16,), jnp.float32)
        return 0
    lax.fori_loop(0, _C2, zrow, 0)
    pltpu.sync_copy(ab0, out_sh.at[pl.ds(s * _RPT2, _C2)])
    pltpu.sync_copy(ab0.at[pl.ds(0, _RPT2 - _C2)],
                    out_sh.at[pl.ds(s * _RPT2 + _C2, _RPT2 - _C2)])
    @pl.when(s == 0)
    def _():
        pltpu.sync_copy(ab0.at[pl.ds(0, 8)], out_sh.at[pl.ds(_HNP, 8)])

    pltpu.sync_copy(denom_hbm, inv_v)

    def invloop(i, _):
        sl = pl.ds(i * 16, 16)
        dsum = inv_v[sl]
        inv_v[sl] = jnp.where(dsum > 0, 1.0 / jnp.where(dsum > 0, dsum, 1.0), 0.0)
        return 0
    lax.fori_loop(0, _NP // 16, invloop, 0)

    def piece(p2, off0):
        pof = p2 * _PCH
        pltpu.sync_copy(idxa_hbm.at[s, pl.ds(pof, _PCH)], idxa_v)
        pltpu.sync_copy(tgt_hbm.at[s, pl.ds(pof, _PCH)], tgt_v)
        pltpu.sync_copy(es_hbm.at[s, pl.ds(pof, _PCH)], ese_v)

        def scan(i, off):
            ci = i // (_C // 16)
            sl = pl.ds((i % (_C // 16)) * 16, 16)
            tg = tgt_v[ci, sl]
            row = tg - c * _HNP
            m = (row >= 0) & (row < _HNP)
            dist = ese_v[ci, sl] * plsc.load_gather(inv_v, [tg])
            plsc.store_compressed(cidx_v.at[pl.ds(off, 16)], idxa_v[ci, sl],
                                  mask=m)
            plsc.store_compressed(cdist_v.at[pl.ds(off, 16)], dist, mask=m)
            plsc.store_compressed(crow_v.at[pl.ds(off, 16)], row, mask=m)
            return off + plsc.all_reduce_population_count(m)[0]
        return lax.fori_loop(0, _PCH * (_C // 16), scan, off0)
    cnt = lax.fori_loop(0, _NCH2 // _PCH, piece, jnp.int32(0))

    for g in range(_C2 // 16):
        sl = pl.ds(cnt + g * 16, 16)
        cidx_v[sl] = jnp.zeros((16,), jnp.int32)
        cdist_v[sl] = jnp.zeros((16,), jnp.float32)
        crow_v[sl] = jnp.full((16,), _HNP, jnp.int32)
    nch = (cnt + _C2 - 1) // _C2
    plsc.subcore_barrier()

    def chunk_body(ci, _):
        base = ci * _C2
        cp0 = pltpu.async_copy(
            aflat_hbm.at[cidx_v.at[pl.ds(base, _C)]],
            abuf_v.at[pl.ds(0, _C)], sem)
        cp1 = pltpu.async_copy(
            aflat_hbm.at[cidx_v.at[pl.ds(base + _C, _C)]],
            abuf_v.at[pl.ds(_C, _C)], sem)
        for g in range(_C // 16):
            rowa_v[pl.ds(g * 16, 16)] = crow_v[pl.ds(base + g * 16, 16)]
            rowb_v[pl.ds(g * 16, 16)] = crow_v[pl.ds(base + _C + g * 16, 16)]
        cp0.wait()
        cp1.wait()

        def grp(g, _):
            sl = pl.ds(base + g * 16, 16)
            dv = cdist_v[sl]
            for l in range(16):
                sc = dv[l]
                e = g * 16 + l
                for h in range(_D // 16):
                    sl2 = pl.ds(h * 16, 16)
                    abuf_v[e, sl2] = abuf_v[e, sl2] * sc
            return 0
        lax.fori_loop(0, _C2 // 16, grp, 0)
        sp0 = pltpu.async_copy(abuf_v.at[pl.ds(0, _C)], out_sh.at[rowa_v],
                               sem, add=True)
        sp1 = pltpu.async_copy(abuf_v.at[pl.ds(_C, _C)], out_sh.at[rowb_v],
                               sem, add=True)
        sp0.wait()
        sp1.wait()
        return 0
    lax.fori_loop(0, nch, chunk_body, 0)

    plsc.subcore_barrier()
    pltpu.sync_copy(out_sh.at[pl.ds(s * _RPT2, _RPT2)],
                    out_hbm.at[pl.ds(c * _HNP + s * _RPT2, _RPT2)])


def kernel(entity_embeds, edges, chunks, edge_type_emb, W_src, b_src, W_msg,
           W_intent, b_intent, W_ibn, b_ibn):
    del chunks
    n = entity_embeds.shape[0]
    e = edges.shape[0]
    ep = _EPAD - e
    src = jnp.pad(edges[:, 0], (0, ep))
    tgt = jnp.pad(edges[:, 1], (0, ep))
    tt = jnp.pad(edges[:, 2], (0, ep))
    et = jnp.pad(edges[:, 3], (0, ep))
    idxv3 = (et * _NP + src).reshape(_NW, _NCH, _C)
    idxu3 = ((tt * _NP + tgt) * _NET + et).reshape(_NW, _NCH, _C)
    tgt3 = tgt.reshape(_NW, _NCH, _C)
    idxa2 = idxv3.reshape(16, _NCH2, _C)
    tgt2 = tgt3.reshape(16, _NCH2, _C)

    x0 = jnp.pad(entity_embeds, ((0, _NP - n), (0, 0)))

    def layer(_, x):
        a, v, u = _tc_pre(x, W_msg, W_src, b_src, edge_type_emb,
                          W_intent, b_intent, W_ibn, b_ibn)
        es, den = _sc_scores(idxv3, tgt3, idxu3, v.reshape(-1),
                             u[:, :, :_NET].reshape(-1))
        return _sc_messages(idxa2, tgt2, es.reshape(16, _NCH2, _C),
                            den[0] + den[1], a.reshape(_NET * _NP, _D))

    x = lax.fori_loop(0, _LAYERS, layer, x0)
    return x[:n]


# speedup vs baseline: 1.1208x; 1.1208x over previous
import functools

import jax
import jax.numpy as jnp
from jax import lax
from jax.experimental import pallas as pl
from jax.experimental.pallas import tpu as pltpu
from jax.experimental.pallas import tpu_sc as plsc

_N = 10000
_NP = 10240
_E = 160000
_D = 128
_DT = 64
_NI = 8
_NET = 9
_NNT = 3
_LAYERS = 3

_NW = 32
_C = 128
_EPAD = 163840
_EPT = _EPAD // _NW
_NCH = _EPT // _C
_RPS = _NP // 16

_BN = 512
_SELU_SCALE = 1.0507009873554805
_SELU_ALPHA = 1.6732632423543772


def _softmax(z):
    m = jnp.max(z, axis=-1, keepdims=True)
    e = jnp.exp(z - m)
    return e / jnp.sum(e, axis=-1, keepdims=True)


def _tc_body(x_ref, wmsg_ref, wsrc_ref, bsrc_ref, etv_ref, wint_ref, bint_ref,
             wibn_ref, bibn_ref, a_ref, v_ref, u_ref):
    xb = x_ref[...]
    for t in range(_NET):
        a_ref[t] = jnp.dot(xb, wmsg_ref[t], preferred_element_type=jnp.float32)
        h = jnp.dot(xb, wsrc_ref[t], preferred_element_type=jnp.float32) + bsrc_ref[t][None, :]
        s = _SELU_SCALE * jnp.where(h > 0, h, _SELU_ALPHA * (jnp.exp(h) - 1.0))
        v_ref[t] = jnp.sum(s * etv_ref[t][None, :], axis=1)
    for t in range(_NNT):
        ti = _softmax(jnp.dot(xb, wint_ref[t], preferred_element_type=jnp.float32) + bint_ref[t][None, :])
        tw = _softmax(jnp.dot(ti, wibn_ref[t], preferred_element_type=jnp.float32) + bibn_ref[t][None, :])
        u_ref[t] = jnp.concatenate(
            [tw, jnp.zeros((tw.shape[0], 128 - _NET), jnp.float32)], axis=1)


_tc_pre = pl.pallas_call(
    _tc_body,
    grid=(_NP // _BN,),
    in_specs=[
        pl.BlockSpec((_BN, _D), lambda i: (i, 0)),
        pl.BlockSpec((_NET, _D, _D), lambda i: (0, 0, 0)),
        pl.BlockSpec((_NET, _D, _DT), lambda i: (0, 0, 0)),
        pl.BlockSpec((_NET, _DT), lambda i: (0, 0)),
        pl.BlockSpec((_NET, _DT), lambda i: (0, 0)),
        pl.BlockSpec((_NNT, _D, _NI), lambda i: (0, 0, 0)),
        pl.BlockSpec((_NNT, _NI), lambda i: (0, 0)),
        pl.BlockSpec((_NNT, _NI, _NET), lambda i: (0, 0, 0)),
        pl.BlockSpec((_NNT, _NET), lambda i: (0, 0)),
    ],
    out_specs=[
        pl.BlockSpec((_NET, _BN, _D), lambda i: (0, i, 0)),
        pl.BlockSpec((_NET, _BN), lambda i: (0, i)),
        pl.BlockSpec((_NNT, _BN, 128), lambda i: (0, i, 0)),
    ],
    out_shape=[
        jax.ShapeDtypeStruct((_NET, _NP, _D), jnp.float32),
        jax.ShapeDtypeStruct((_NET, _NP), jnp.float32),
        jax.ShapeDtypeStruct((_NNT, _NP, 128), jnp.float32),
    ],
)

_sc_mesh = plsc.VectorSubcoreMesh(core_axis_name="c", subcore_axis_name="s")
_sc_params = pltpu.CompilerParams(needs_layout_passes=False)


_USZ = _NNT * _NP * _NET
_UPT = _USZ // 16


@functools.partial(
    pl.kernel,
    mesh=_sc_mesh,
    compiler_params=_sc_params,
    out_type=[
        jax.ShapeDtypeStruct((_NW, _NCH, _C), jnp.float32),
        jax.ShapeDtypeStruct((2, _NP), jnp.float32),
    ],
    scratch_types=[
        pltpu.VMEM((_NET * _NP,), jnp.float32),
        pltpu.VMEM((_NCH, _C), jnp.int32),
        pltpu.VMEM((_NCH, _C), jnp.int32),
        pltpu.VMEM((_NCH, _C), jnp.int32),
        pltpu.VMEM((_C,), jnp.float32),
        pltpu.VMEM((_C,), jnp.float32),
        pltpu.VMEM((_NCH, _C), jnp.float32),
        pltpu.VMEM((_RPS,), jnp.float32),
        pltpu.VMEM_SHARED((_NP,), jnp.float32),
        pltpu.SemaphoreType.DMA,
        pltpu.SemaphoreType.DMA,
    ],
)
def _sc_scores(idxv_hbm, tgt_hbm, idxu_hbm, vflat_hbm, u9_hbm,
               es_hbm, denom_hbm,
               vflat_v, idxv_v, tgt_v, idxu_v, ubuf0_v, ubuf1_v, es_v,
               zero_v, denom_sh, g0, g1):
    c = lax.axis_index("c")
    s = lax.axis_index("s")
    wid = c * 16 + s

    def zloop(i, _):
        zero_v[pl.ds(i * 16, 16)] = jnp.zeros((16,), jnp.float32)
        return 0
    lax.fori_loop(0, _RPS // 16, zloop, 0)
    pltpu.sync_copy(zero_v, denom_sh.at[pl.ds(s * _RPS, _RPS)])

    pltpu.sync_copy(vflat_hbm, vflat_v)
    pltpu.sync_copy(idxv_hbm.at[wid], idxv_v)
    pltpu.sync_copy(tgt_hbm.at[wid], tgt_v)
    pltpu.sync_copy(idxu_hbm.at[wid], idxu_v)
    plsc.subcore_barrier()

    pltpu.async_copy(u9_hbm.at[idxu_v.at[0]], ubuf0_v, g0)
    pltpu.async_copy(u9_hbm.at[idxu_v.at[1]], ubuf1_v, g1)

    def finish(ci, ubuf, gsem):
        pltpu.make_async_copy(u9_hbm.at[idxu_v.at[ci]], ubuf, gsem).wait()
        for g in range(_C // 16):
            sl = pl.ds(g * 16, 16)
            v = plsc.load_gather(vflat_v, [idxv_v[ci, sl]])
            u = ubuf[sl]
            gid = (wid * _EPT + ci * _C + g * 16) + lax.iota(jnp.int32, 16)
            es_v[ci, sl] = jnp.where(gid < _E, jnp.exp(v * u), 0.0)
        pltpu.sync_copy(es_v.at[ci], denom_sh.at[tgt_v.at[ci]], add=True)

    def pair(p, _):
        finish(2 * p, ubuf0_v, g0)

        @pl.when(p < _NCH // 2 - 1)
        def _():
            pltpu.async_copy(u9_hbm.at[idxu_v.at[2 * p + 2]], ubuf0_v, g0)
        finish(2 * p + 1, ubuf1_v, g1)

        @pl.when(p < _NCH // 2 - 1)
        def _():
            pltpu.async_copy(u9_hbm.at[idxu_v.at[2 * p + 3]], ubuf1_v, g1)
        return 0
    lax.fori_loop(0, _NCH // 2, pair, 0)

    pltpu.sync_copy(es_v, es_hbm.at[wid])
    plsc.subcore_barrier()
    pltpu.sync_copy(denom_sh.at[pl.ds(s * _RPS, _RPS)],
                    denom_hbm.at[c, pl.ds(s * _RPS, _RPS)])


_HNP = _NP // 2
_EPT2 = _EPAD // 16
_NCH2 = _EPT2 // _C
_RPT2 = _HNP // 16


@functools.partial(
    pl.kernel,
    mesh=_sc_mesh,
    compiler_params=_sc_params,
    out_type=jax.ShapeDtypeStruct((_NP, _D), jnp.float32),
    scratch_types=[
        pltpu.VMEM((_NP,), jnp.float32),
        pltpu.VMEM((_NCH2, _C), jnp.int32),
        pltpu.VMEM((_NCH2, _C), jnp.int32),
        pltpu.VMEM((_NCH2, _C), jnp.float32),
        pltpu.VMEM((_EPT2 + _C,), jnp.int32),
        pltpu.VMEM((_EPT2 + _C,), jnp.float32),
        pltpu.VMEM((_EPT2 + _C,), jnp.int32),
        pltpu.VMEM((_C,), jnp.int32),
        pltpu.VMEM((_C,), jnp.int32),
        pltpu.VMEM((_C, _D), jnp.float32),
        pltpu.VMEM_SHARED((_HNP + 8, _D), jnp.float32),
        pltpu.SemaphoreType.DMA,
    ],
)
def _sc_messages(idxa_hbm, tgt_hbm, es_hbm, denom_hbm, aflat_hbm,
                 out_hbm,
                 inv_v, idxa_v, tgt_v, ese_v, cidx_v, cdist_v,
                 crow_v, row2_v, idx2_v, abuf_v, out_sh, sem):
    c = lax.axis_index("c")
    s = lax.axis_index("s")
    ab0 = abuf_v

    def zrow(i, _):
        for h in range(_D // 16):
            ab0[i, pl.ds(h * 16, 16)] = jnp.zeros((16,), jnp.float32)
        return 0
    lax.fori_loop(0, _C, zrow, 0)
    pltpu.sync_copy(ab0, out_sh.at[pl.ds(s * _RPT2, _C)])
    pltpu.sync_copy(ab0, out_sh.at[pl.ds(s * _RPT2 + _C, _C)])
    pltpu.sync_copy(ab0.at[pl.ds(0, _RPT2 - 2 * _C)],
                    out_sh.at[pl.ds(s * _RPT2 + 2 * _C, _RPT2 - 2 * _C)])
    @pl.when(s == 0)
    def _():
        pltpu.sync_copy(ab0.at[pl.ds(0, 8)], out_sh.at[pl.ds(_HNP, 8)])

    pltpu.sync_copy(denom_hbm, inv_v)

    def invloop(i, _):
        sl = pl.ds(i * 16, 16)
        dsum = inv_v[sl]
        inv_v[sl] = jnp.where(dsum > 0, 1.0 / jnp.where(dsum > 0, dsum, 1.0), 0.0)
        return 0
    lax.fori_loop(0, _NP // 16, invloop, 0)

    pltpu.sync_copy(idxa_hbm.at[s], idxa_v)
    pltpu.sync_copy(tgt_hbm.at[s], tgt_v)
    pltpu.sync_copy(es_hbm.at[s], ese_v)

    def scan(i, off):
        ci = i // (_C // 16)
        sl = pl.ds((i % (_C // 16)) * 16, 16)
        tg = tgt_v[ci, sl]
        row = tg - c * _HNP
        m = (row >= 0) & (row < _HNP)
        dist = ese_v[ci, sl] * plsc.load_gather(inv_v, [tg])
        plsc.store_compressed(cidx_v.at[pl.ds(off, 16)], idxa_v[ci, sl], mask=m)
        plsc.store_compressed(cdist_v.at[pl.ds(off, 16)], dist, mask=m)
        plsc.store_compressed(crow_v.at[pl.ds(off, 16)], row, mask=m)
        return off + plsc.all_reduce_population_count(m)[0]
    cnt = lax.fori_loop(0, _NCH2 * (_C // 16), scan, jnp.int32(0))

    for g in range(_C // 16):
        sl = pl.ds(cnt + g * 16, 16)
        cidx_v[sl] = jnp.zeros((16,), jnp.int32)
        cdist_v[sl] = jnp.zeros((16,), jnp.float32)
        crow_v[sl] = jnp.full((16,), _HNP, jnp.int32)
    nch = (cnt + _C - 1) // _C
    plsc.subcore_barrier()

    def chunk_body(ci, _):
        base = ci * _C
        for g in range(_C // 16):
            idx2_v[pl.ds(g * 16, 16)] = cidx_v[pl.ds(base + g * 16, 16)]
        pltpu.async_copy(aflat_hbm.at[idx2_v], abuf_v, sem).wait()

        def grp(g, _):
            sl = pl.ds(base + g * 16, 16)
            row2_v[pl.ds(g * 16, 16)] = crow_v[sl]
            dv = cdist_v[sl]
            for l in range(16):
                sc = dv[l]
                e = g * 16 + l
                for h in range(_D // 16):
                    sl2 = pl.ds(h * 16, 16)
                    abuf_v[e, sl2] = abuf_v[e, sl2] * sc
            return 0
        lax.fori_loop(0, _C // 16, grp, 0)
        pltpu.sync_copy(abuf_v, out_sh.at[row2_v], add=True)
        return 0
    lax.fori_loop(0, nch, chunk_body, 0)

    plsc.subcore_barrier()
    pltpu.sync_copy(out_sh.at[pl.ds(s * _RPT2, _RPT2)],
                    out_hbm.at[pl.ds(c * _HNP + s * _RPT2, _RPT2)])


def kernel(entity_embeds, edges, chunks, edge_type_emb, W_src, b_src, W_msg,
           W_intent, b_intent, W_ibn, b_ibn):
    del chunks
    n = entity_embeds.shape[0]
    e = edges.shape[0]
    ep = _EPAD - e
    src = jnp.pad(edges[:, 0], (0, ep))
    tgt = jnp.pad(edges[:, 1], (0, ep))
    tt = jnp.pad(edges[:, 2], (0, ep))
    et = jnp.pad(edges[:, 3], (0, ep))
    idxv3 = (et * _NP + src).reshape(_NW, _NCH, _C)
    idxu3 = ((tt * _NP + tgt) * _NET + et).reshape(_NW, _NCH, _C)
    tgt3 = tgt.reshape(_NW, _NCH, _C)
    idxa2 = idxv3.reshape(16, _NCH2, _C)
    tgt2 = tgt3.reshape(16, _NCH2, _C)

    x0 = jnp.pad(entity_embeds, ((0, _NP - n), (0, 0)))

    def layer(_, x):
        a, v, u = _tc_pre(x, W_msg, W_src, b_src, edge_type_emb,
                          W_intent, b_intent, W_ibn, b_ibn)
        es, den = _sc_scores(idxv3, tgt3, idxu3, v.reshape(-1),
                             u[:, :, :_NET].reshape(-1))
        return _sc_messages(idxa2, tgt2, es.reshape(16, _NCH2, _C),
                            den[0] + den[1], a.reshape(_NET * _NP, _D))

    x = lax.fori_loop(0, _LAYERS, layer, x0)
    return x[:n]


# speedup vs baseline: 1.1234x; 1.0023x over previous
import functools

import jax
import jax.numpy as jnp
from jax import lax
from jax.experimental import pallas as pl
from jax.experimental.pallas import tpu as pltpu
from jax.experimental.pallas import tpu_sc as plsc

_N = 10000
_NP = 10240
_E = 160000
_D = 128
_DT = 64
_NI = 8
_NET = 9
_NNT = 3
_LAYERS = 3

_NW = 32
_C = 128
_EPAD = 163840
_EPT = _EPAD // _NW
_NCH = _EPT // _C
_RPS = _NP // 16

_BN = 512
_SELU_SCALE = 1.0507009873554805
_SELU_ALPHA = 1.6732632423543772


def _softmax(z):
    m = jnp.max(z, axis=-1, keepdims=True)
    e = jnp.exp(z - m)
    return e / jnp.sum(e, axis=-1, keepdims=True)


def _tc_body(x_ref, wmsg_ref, wsrc_ref, bsrc_ref, etv_ref, wint_ref, bint_ref,
             wibn_ref, bibn_ref, a_ref, v_ref, u_ref):
    xb = x_ref[...]
    for t in range(_NET):
        a_ref[t] = jnp.dot(xb, wmsg_ref[t], preferred_element_type=jnp.float32)
        h = jnp.dot(xb, wsrc_ref[t], preferred_element_type=jnp.float32) + bsrc_ref[t][None, :]
        s = _SELU_SCALE * jnp.where(h > 0, h, _SELU_ALPHA * (jnp.exp(h) - 1.0))
        v_ref[t] = jnp.sum(s * etv_ref[t][None, :], axis=1)
    for t in range(_NNT):
        ti = _softmax(jnp.dot(xb, wint_ref[t], preferred_element_type=jnp.float32) + bint_ref[t][None, :])
        tw = _softmax(jnp.dot(ti, wibn_ref[t], preferred_element_type=jnp.float32) + bibn_ref[t][None, :])
        u_ref[t] = jnp.concatenate(
            [tw, jnp.zeros((tw.shape[0], 128 - _NET), jnp.float32)], axis=1)


_tc_pre = pl.pallas_call(
    _tc_body,
    grid=(_NP // _BN,),
    in_specs=[
        pl.BlockSpec((_BN, _D), lambda i: (i, 0)),
        pl.BlockSpec((_NET, _D, _D), lambda i: (0, 0, 0)),
        pl.BlockSpec((_NET, _D, _DT), lambda i: (0, 0, 0)),
        pl.BlockSpec((_NET, _DT), lambda i: (0, 0)),
        pl.BlockSpec((_NET, _DT), lambda i: (0, 0)),
        pl.BlockSpec((_NNT, _D, _NI), lambda i: (0, 0, 0)),
        pl.BlockSpec((_NNT, _NI), lambda i: (0, 0)),
        pl.BlockSpec((_NNT, _NI, _NET), lambda i: (0, 0, 0)),
        pl.BlockSpec((_NNT, _NET), lambda i: (0, 0)),
    ],
    out_specs=[
        pl.BlockSpec((_NET, _BN, _D), lambda i: (0, i, 0)),
        pl.BlockSpec((_NET, _BN), lambda i: (0, i)),
        pl.BlockSpec((_NNT, _BN, 128), lambda i: (0, i, 0)),
    ],
    out_shape=[
        jax.ShapeDtypeStruct((_NET, _NP, _D), jnp.float32),
        jax.ShapeDtypeStruct((_NET, _NP), jnp.float32),
        jax.ShapeDtypeStruct((_NNT, _NP, 128), jnp.float32),
    ],
)

_sc_mesh = plsc.VectorSubcoreMesh(core_axis_name="c", subcore_axis_name="s")
_sc_params = pltpu.CompilerParams(needs_layout_passes=False)


_USZ = _NNT * _NP * _NET
_UPT = _USZ // 16


@functools.partial(
    pl.kernel,
    mesh=_sc_mesh,
    compiler_params=_sc_params,
    out_type=[
        jax.ShapeDtypeStruct((_NW, _NCH, _C), jnp.float32),
        jax.ShapeDtypeStruct((2, _NP), jnp.float32),
    ],
    scratch_types=[
        pltpu.VMEM((_NET * _NP,), jnp.float32),
        pltpu.VMEM((_NCH, _C), jnp.int32),
        pltpu.VMEM((_NCH, _C), jnp.int32),
        pltpu.VMEM((_NCH, _C), jnp.int32),
        pltpu.VMEM((_C,), jnp.float32),
        pltpu.VMEM((_C,), jnp.float32),
        pltpu.VMEM((_NCH, _C), jnp.float32),
        pltpu.VMEM((_RPS,), jnp.float32),
        pltpu.VMEM_SHARED((_NP,), jnp.float32),
        pltpu.SemaphoreType.DMA,
        pltpu.SemaphoreType.DMA,
    ],
)
def _sc_scores(idxv_hbm, tgt_hbm, idxu_hbm, vflat_hbm, u9_hbm,
               es_hbm, denom_hbm,
               vflat_v, idxv_v, tgt_v, idxu_v, ubuf0_v, ubuf1_v, es_v,
               zero_v, denom_sh, g0, g1):
    c = lax.axis_index("c")
    s = lax.axis_index("s")
    wid = c * 16 + s

    def zloop(i, _):
        zero_v[pl.ds(i * 16, 16)] = jnp.zeros((16,), jnp.float32)
        return 0
    lax.fori_loop(0, _RPS // 16, zloop, 0)
    pltpu.sync_copy(zero_v, denom_sh.at[pl.ds(s * _RPS, _RPS)])

    pltpu.sync_copy(vflat_hbm, vflat_v)
    pltpu.sync_copy(idxv_hbm.at[wid], idxv_v)
    pltpu.sync_copy(tgt_hbm.at[wid], tgt_v)
    pltpu.sync_copy(idxu_hbm.at[wid], idxu_v)
    plsc.subcore_barrier()

    pltpu.async_copy(u9_hbm.at[idxu_v.at[0]], ubuf0_v, g0)
    pltpu.async_copy(u9_hbm.at[idxu_v.at[1]], ubuf1_v, g1)

    def finish(ci, ubuf, gsem):
        pltpu.make_async_copy(u9_hbm.at[idxu_v.at[ci]], ubuf, gsem).wait()
        for g in range(_C // 16):
            sl = pl.ds(g * 16, 16)
            v = plsc.load_gather(vflat_v, [idxv_v[ci, sl]])
            u = ubuf[sl]
            gid = (wid * _EPT + ci * _C + g * 16) + lax.iota(jnp.int32, 16)
            es_v[ci, sl] = jnp.where(gid < _E, jnp.exp(v * u), 0.0)
        pltpu.sync_copy(es_v.at[ci], denom_sh.at[tgt_v.at[ci]], add=True)

    def pair(p, _):
        finish(2 * p, ubuf0_v, g0)

        @pl.when(p < _NCH // 2 - 1)
        def _():
            pltpu.async_copy(u9_hbm.at[idxu_v.at[2 * p + 2]], ubuf0_v, g0)
        finish(2 * p + 1, ubuf1_v, g1)

        @pl.when(p < _NCH // 2 - 1)
        def _():
            pltpu.async_copy(u9_hbm.at[idxu_v.at[2 * p + 3]], ubuf1_v, g1)
        return 0
    lax.fori_loop(0, _NCH // 2, pair, 0)

    pltpu.sync_copy(es_v, es_hbm.at[wid])
    plsc.subcore_barrier()
    pltpu.sync_copy(denom_sh.at[pl.ds(s * _RPS, _RPS)],
                    denom_hbm.at[c, pl.ds(s * _RPS, _RPS)])


_HNP = _NP // 2
_EPT2 = _EPAD // 16
_NCH2 = _EPT2 // _C
_RPT2 = _HNP // 16


@functools.partial(
    pl.kernel,
    mesh=_sc_mesh,
    compiler_params=_sc_params,
    out_type=jax.ShapeDtypeStruct((_NP, _D), jnp.float32),
    scratch_types=[
        pltpu.VMEM((_NP,), jnp.float32),
        pltpu.VMEM((_NCH2, _C), jnp.int32),
        pltpu.VMEM((_NCH2, _C), jnp.int32),
        pltpu.VMEM((_NCH2, _C), jnp.float32),
        pltpu.VMEM((_EPT2 + _C,), jnp.int32),
        pltpu.VMEM((_EPT2 + _C,), jnp.float32),
        pltpu.VMEM((_EPT2 + _C,), jnp.int32),
        pltpu.VMEM((_C,), jnp.int32),
        pltpu.VMEM((_C,), jnp.int32),
        pltpu.VMEM((_C, _D), jnp.float32),
        pltpu.VMEM_SHARED((_HNP + 8, _D), jnp.float32),
        pltpu.SemaphoreType.DMA,
    ],
)
def _sc_messages(idxa_hbm, tgt_hbm, es_hbm, denom_hbm, aflat_hbm,
                 out_hbm,
                 inv_v, idxa_v, tgt_v, ese_v, cidx_v, cdist_v,
                 crow_v, row2_v, idx2_v, abuf_v, out_sh, sem):
    c = lax.axis_index("c")
    s = lax.axis_index("s")
    ab0 = abuf_v

    def zrow(i, _):
        for h in range(_D // 16):
            ab0[i, pl.ds(h * 16, 16)] = jnp.zeros((16,), jnp.float32)
        return 0
    lax.fori_loop(0, _C, zrow, 0)
    pltpu.sync_copy(ab0, out_sh.at[pl.ds(s * _RPT2, _C)])
    pltpu.sync_copy(ab0, out_sh.at[pl.ds(s * _RPT2 + _C, _C)])
    pltpu.sync_copy(ab0.at[pl.ds(0, _RPT2 - 2 * _C)],
                    out_sh.at[pl.ds(s * _RPT2 + 2 * _C, _RPT2 - 2 * _C)])
    @pl.when(s == 0)
    def _():
        pltpu.sync_copy(ab0.at[pl.ds(0, 8)], out_sh.at[pl.ds(_HNP, 8)])

    pltpu.sync_copy(denom_hbm, inv_v)

    def invloop(i, _):
        sl = pl.ds(i * 16, 16)
        dsum = inv_v[sl]
        inv_v[sl] = jnp.where(dsum > 0, 1.0 / jnp.where(dsum > 0, dsum, 1.0), 0.0)
        return 0
    lax.fori_loop(0, _NP // 16, invloop, 0)

    pltpu.sync_copy(idxa_hbm.at[s], idxa_v)
    pltpu.sync_copy(tgt_hbm.at[s], tgt_v)
    pltpu.sync_copy(es_hbm.at[s], ese_v)

    def scan(i, off):
        ci = i // (_C // 16)
        sl = pl.ds((i % (_C // 16)) * 16, 16)
        tg = tgt_v[ci, sl]
        row = tg - c * _HNP
        m = (row >= 0) & (row < _HNP)
        dist = ese_v[ci, sl] * plsc.load_gather(inv_v, [tg])
        plsc.store_compressed(cidx_v.at[pl.ds(off, 16)], idxa_v[ci, sl], mask=m)
        plsc.store_compressed(cdist_v.at[pl.ds(off, 16)], dist, mask=m)
        plsc.store_compressed(crow_v.at[pl.ds(off, 16)], row, mask=m)
        return off + plsc.all_reduce_population_count(m)[0]
    cnt = lax.fori_loop(0, _NCH2 * (_C // 16), scan, jnp.int32(0))

    for g in range(_C // 16):
        sl = pl.ds(cnt + g * 16, 16)
        cidx_v[sl] = jnp.zeros((16,), jnp.int32)
        cdist_v[sl] = jnp.zeros((16,), jnp.float32)
        crow_v[sl] = jnp.full((16,), _HNP, jnp.int32)
    nch = (cnt + _C - 1) // _C
    plsc.subcore_barrier()

    def chunk_body(ci, _):
        base = ci * _C
        pltpu.async_copy(aflat_hbm.at[cidx_v.at[pl.ds(base, _C)]],
                         abuf_v, sem).wait()

        def grp(g, _):
            sl = pl.ds(base + g * 16, 16)
            row2_v[pl.ds(g * 16, 16)] = crow_v[sl]
            dv = cdist_v[sl]
            for l in range(16):
                sc = dv[l]
                e = g * 16 + l
                for h in range(_D // 16):
                    sl2 = pl.ds(h * 16, 16)
                    abuf_v[e, sl2] = abuf_v[e, sl2] * sc
            return 0
        lax.fori_loop(0, _C // 16, grp, 0)
        pltpu.sync_copy(abuf_v, out_sh.at[row2_v], add=True)
        return 0
    lax.fori_loop(0, nch, chunk_body, 0)

    plsc.subcore_barrier()
    pltpu.sync_copy(out_sh.at[pl.ds(s * _RPT2, _RPT2)],
                    out_hbm.at[pl.ds(c * _HNP + s * _RPT2, _RPT2)])


def kernel(entity_embeds, edges, chunks, edge_type_emb, W_src, b_src, W_msg,
           W_intent, b_intent, W_ibn, b_ibn):
    del chunks
    n = entity_embeds.shape[0]
    e = edges.shape[0]
    ep = _EPAD - e
    src = jnp.pad(edges[:, 0], (0, ep))
    tgt = jnp.pad(edges[:, 1], (0, ep))
    tt = jnp.pad(edges[:, 2], (0, ep))
    et = jnp.pad(edges[:, 3], (0, ep))
    idxv3 = (et * _NP + src).reshape(_NW, _NCH, _C)
    idxu3 = ((tt * _NP + tgt) * _NET + et).reshape(_NW, _NCH, _C)
    tgt3 = tgt.reshape(_NW, _NCH, _C)
    idxa2 = idxv3.reshape(16, _NCH2, _C)
    tgt2 = tgt3.reshape(16, _NCH2, _C)

    x0 = jnp.pad(entity_embeds, ((0, _NP - n), (0, 0)))

    def layer(_, x):
        a, v, u = _tc_pre(x, W_msg, W_src, b_src, edge_type_emb,
                          W_intent, b_intent, W_ibn, b_ibn)
        es, den = _sc_scores(idxv3, tgt3, idxu3, v.reshape(-1),
                             u[:, :, :_NET].reshape(-1))
        return _sc_messages(idxa2, tgt2, es.reshape(16, _NCH2, _C),
                            den[0] + den[1], a.reshape(_NET * _NP, _D))

    x = lax.fori_loop(0, _LAYERS, layer, x0)
    return x[:n]


# speedup vs baseline: 1.1234x; 1.0000x over previous
import functools

import jax
import jax.numpy as jnp
from jax import lax
from jax.experimental import pallas as pl
from jax.experimental.pallas import tpu as pltpu
from jax.experimental.pallas import tpu_sc as plsc

_N = 10000
_NP = 10240
_E = 160000
_D = 128
_DT = 64
_NI = 8
_NET = 9
_NNT = 3
_LAYERS = 3

_NW = 32
_C = 128
_EPAD = 163840
_EPT = _EPAD // _NW
_NCH = _EPT // _C
_RPS = _NP // 16

_BN = 512
_SELU_SCALE = 1.0507009873554805
_SELU_ALPHA = 1.6732632423543772


def _softmax(z):
    m = jnp.max(z, axis=-1, keepdims=True)
    e = jnp.exp(z - m)
    return e / jnp.sum(e, axis=-1, keepdims=True)


def _tc_body(x_ref, wmsg_ref, wsrc_ref, bsrc_ref, etv_ref, wint_ref, bint_ref,
             wibn_ref, bibn_ref, a_ref, v_ref, u_ref):
    xb = x_ref[...]
    for t in range(_NET):
        a_ref[t] = jnp.dot(xb, wmsg_ref[t], preferred_element_type=jnp.float32)
        h = jnp.dot(xb, wsrc_ref[t], preferred_element_type=jnp.float32) + bsrc_ref[t][None, :]
        s = _SELU_SCALE * jnp.where(h > 0, h, _SELU_ALPHA * (jnp.exp(h) - 1.0))
        v_ref[t] = jnp.sum(s * etv_ref[t][None, :], axis=1)
    for t in range(_NNT):
        ti = _softmax(jnp.dot(xb, wint_ref[t], preferred_element_type=jnp.float32) + bint_ref[t][None, :])
        tw = _softmax(jnp.dot(ti, wibn_ref[t], preferred_element_type=jnp.float32) + bibn_ref[t][None, :])
        u_ref[t] = jnp.concatenate(
            [tw, jnp.zeros((tw.shape[0], 128 - _NET), jnp.float32)], axis=1)


_tc_pre = pl.pallas_call(
    _tc_body,
    grid=(_NP // _BN,),
    in_specs=[
        pl.BlockSpec((_BN, _D), lambda i: (i, 0)),
        pl.BlockSpec((_NET, _D, _D), lambda i: (0, 0, 0)),
        pl.BlockSpec((_NET, _D, _DT), lambda i: (0, 0, 0)),
        pl.BlockSpec((_NET, _DT), lambda i: (0, 0)),
        pl.BlockSpec((_NET, _DT), lambda i: (0, 0)),
        pl.BlockSpec((_NNT, _D, _NI), lambda i: (0, 0, 0)),
        pl.BlockSpec((_NNT, _NI), lambda i: (0, 0)),
        pl.BlockSpec((_NNT, _NI, _NET), lambda i: (0, 0, 0)),
        pl.BlockSpec((_NNT, _NET), lambda i: (0, 0)),
    ],
    out_specs=[
        pl.BlockSpec((_NET, _BN, _D), lambda i: (0, i, 0)),
        pl.BlockSpec((_NET, _BN), lambda i: (0, i)),
        pl.BlockSpec((_NNT, _BN, 128), lambda i: (0, i, 0)),
    ],
    out_shape=[
        jax.ShapeDtypeStruct((_NET, _NP, _D), jnp.float32),
        jax.ShapeDtypeStruct((_NET, _NP), jnp.float32),
        jax.ShapeDtypeStruct((_NNT, _NP, 128), jnp.float32),
    ],
)

_sc_mesh = plsc.VectorSubcoreMesh(core_axis_name="c", subcore_axis_name="s")
_sc_params = pltpu.CompilerParams(needs_layout_passes=False)


@functools.partial(
    pl.kernel,
    mesh=_sc_mesh,
    compiler_params=_sc_params,
    out_type=[
        jax.ShapeDtypeStruct((_NW, _NCH, _C), jnp.float32),
        jax.ShapeDtypeStruct((2, _NP), jnp.float32),
    ],
    scratch_types=[
        pltpu.VMEM((_NET * _NP,), jnp.float32),
        pltpu.VMEM((_NCH, _C), jnp.int32),
        pltpu.VMEM((_NCH, _C), jnp.int32),
        pltpu.VMEM((_NCH, _C), jnp.int32),
        pltpu.VMEM((_C,), jnp.float32),
        pltpu.VMEM((_C,), jnp.float32),
        pltpu.VMEM((_NCH, _C), jnp.float32),
        pltpu.VMEM((_RPS,), jnp.float32),
        pltpu.VMEM_SHARED((_NP,), jnp.float32),
        pltpu.SemaphoreType.DMA,
        pltpu.SemaphoreType.DMA,
    ],
)
def _sc_scores(idxv_hbm, tgt_hbm, idxu_hbm, vflat_hbm, u9_hbm,
               es_hbm, denom_hbm,
               vflat_v, idxv_v, tgt_v, idxu_v, ubuf0_v, ubuf1_v, es_v,
               zero_v, denom_sh, g0, g1):
    c = lax.axis_index("c")
    s = lax.axis_index("s")
    wid = c * 16 + s

    def zloop(i, _):
        zero_v[pl.ds(i * 16, 16)] = jnp.zeros((16,), jnp.float32)
        return 0
    lax.fori_loop(0, _RPS // 16, zloop, 0)
    pltpu.sync_copy(zero_v, denom_sh.at[pl.ds(s * _RPS, _RPS)])

    pltpu.sync_copy(vflat_hbm, vflat_v)
    pltpu.sync_copy(idxv_hbm.at[wid], idxv_v)
    pltpu.sync_copy(tgt_hbm.at[wid], tgt_v)
    pltpu.sync_copy(idxu_hbm.at[wid], idxu_v)
    plsc.subcore_barrier()

    pltpu.async_copy(u9_hbm.at[idxu_v.at[0]], ubuf0_v, g0)
    pltpu.async_copy(u9_hbm.at[idxu_v.at[1]], ubuf1_v, g1)

    def finish(ci, ubuf, gsem):
        pltpu.make_async_copy(u9_hbm.at[idxu_v.at[ci]], ubuf, gsem).wait()
        for g in range(_C // 16):
            sl = pl.ds(g * 16, 16)
            v = plsc.load_gather(vflat_v, [idxv_v[ci, sl]])
            u = ubuf[sl]
            gid = (wid * _EPT + ci * _C + g * 16) + lax.iota(jnp.int32, 16)
            es_v[ci, sl] = jnp.where(gid < _E, jnp.exp(v * u), 0.0)
        pltpu.sync_copy(es_v.at[ci], denom_sh.at[tgt_v.at[ci]], add=True)

    def pair(p, _):
        finish(2 * p, ubuf0_v, g0)

        @pl.when(p < _NCH // 2 - 1)
        def _():
            pltpu.async_copy(u9_hbm.at[idxu_v.at[2 * p + 2]], ubuf0_v, g0)
        finish(2 * p + 1, ubuf1_v, g1)

        @pl.when(p < _NCH // 2 - 1)
        def _():
            pltpu.async_copy(u9_hbm.at[idxu_v.at[2 * p + 3]], ubuf1_v, g1)
        return 0
    lax.fori_loop(0, _NCH // 2, pair, 0)

    pltpu.sync_copy(es_v, es_hbm.at[wid])
    plsc.subcore_barrier()
    pltpu.sync_copy(denom_sh.at[pl.ds(s * _RPS, _RPS)],
                    denom_hbm.at[c, pl.ds(s * _RPS, _RPS)])


_HNP = _NP // 2
_EPT2 = _EPAD // 16
_NCH2 = _EPT2 // _C
_RPT2 = _HNP // 16


@functools.partial(
    pl.kernel,
    mesh=_sc_mesh,
    compiler_params=_sc_params,
    out_type=jax.ShapeDtypeStruct((_NP, _D), jnp.float32),
    scratch_types=[
        pltpu.VMEM((_NP,), jnp.float32),
        pltpu.VMEM((_NCH2, _C), jnp.int32),
        pltpu.VMEM((_NCH2, _C), jnp.int32),
        pltpu.VMEM((_NCH2, _C), jnp.float32),
        pltpu.VMEM((_EPT2 + _C,), jnp.int32),
        pltpu.VMEM((_EPT2 + _C,), jnp.float32),
        pltpu.VMEM((_EPT2 + _C,), jnp.int32),
        pltpu.VMEM((_C,), jnp.int32),
        pltpu.VMEM((_C, _D), jnp.float32),
        pltpu.VMEM_SHARED((_HNP + 8, _D), jnp.float32),
        pltpu.SemaphoreType.DMA,
    ],
)
def _sc_messages(idxa_hbm, tgt_hbm, es_hbm, denom_hbm, aflat_hbm,
                 out_hbm,
                 inv_v, idxa_v, tgt_v, ese_v, cidx_v, cdist_v,
                 crow_v, row2_v, abuf_v, out_sh, sem):
    c = lax.axis_index("c")
    s = lax.axis_index("s")
    ab0 = abuf_v

    def zrow(i, _):
        for h in range(_D // 16):
            ab0[i, pl.ds(h * 16, 16)] = jnp.zeros((16,), jnp.float32)
        return 0
    lax.fori_loop(0, _C, zrow, 0)
    pltpu.sync_copy(ab0, out_sh.at[pl.ds(s * _RPT2, _C)])
    pltpu.sync_copy(ab0, out_sh.at[pl.ds(s * _RPT2 + _C, _C)])
    pltpu.sync_copy(ab0.at[pl.ds(0, _RPT2 - 2 * _C)],
                    out_sh.at[pl.ds(s * _RPT2 + 2 * _C, _RPT2 - 2 * _C)])
    @pl.when(s == 0)
    def _():
        pltpu.sync_copy(ab0.at[pl.ds(0, 8)], out_sh.at[pl.ds(_HNP, 8)])

    pltpu.sync_copy(denom_hbm, inv_v)

    def invloop(i, _):
        sl = pl.ds(i * 16, 16)
        dsum = inv_v[sl]
        inv_v[sl] = jnp.where(dsum > 0, 1.0 / jnp.where(dsum > 0, dsum, 1.0), 0.0)
        return 0
    lax.fori_loop(0, _NP // 16, invloop, 0)

    pltpu.sync_copy(idxa_hbm.at[s], idxa_v)
    pltpu.sync_copy(tgt_hbm.at[s], tgt_v)
    pltpu.sync_copy(es_hbm.at[s], ese_v)

    def scan(i, off):
        ci = i // (_C // 16)
        sl = pl.ds((i % (_C // 16)) * 16, 16)
        tg = tgt_v[ci, sl]
        row = tg - c * _HNP
        m = (row >= 0) & (row < _HNP)
        dist = ese_v[ci, sl] * plsc.load_gather(inv_v, [tg])
        plsc.store_compressed(cidx_v.at[pl.ds(off, 16)], idxa_v[ci, sl], mask=m)
        plsc.store_compressed(cdist_v.at[pl.ds(off, 16)], dist, mask=m)
        plsc.store_compressed(crow_v.at[pl.ds(off, 16)], row, mask=m)
        return off + plsc.all_reduce_population_count(m)[0]
    cnt = lax.fori_loop(0, _NCH2 * (_C // 16), scan, jnp.int32(0))

    for g in range(_C // 16):
        sl = pl.ds(cnt + g * 16, 16)
        cidx_v[sl] = jnp.zeros((16,), jnp.int32)
        cdist_v[sl] = jnp.zeros((16,), jnp.float32)
        crow_v[sl] = jnp.full((16,), _HNP, jnp.int32)
    nch = (cnt + _C - 1) // _C
    plsc.subcore_barrier()

    def chunk_body(ci, _):
        base = ci * _C
        pltpu.async_copy(aflat_hbm.at[cidx_v.at[pl.ds(base, _C)]],
                         abuf_v, sem).wait()

        def grp(g, _):
            sl = pl.ds(base + g * 16, 16)
            row2_v[pl.ds(g * 16, 16)] = crow_v[sl]
            dv = cdist_v[sl]
            for l in range(16):
                sc = dv[l]
                e = g * 16 + l
                for h in range(_D // 16):
                    sl2 = pl.ds(h * 16, 16)
                    abuf_v[e, sl2] = abuf_v[e, sl2] * sc
            return 0
        lax.fori_loop(0, _C // 16, grp, 0)
        pltpu.sync_copy(abuf_v, out_sh.at[row2_v], add=True)
        return 0
    lax.fori_loop(0, nch, chunk_body, 0)

    plsc.subcore_barrier()
    pltpu.sync_copy(out_sh.at[pl.ds(s * _RPT2, _RPT2)],
                    out_hbm.at[pl.ds(c * _HNP + s * _RPT2, _RPT2)])


def kernel(entity_embeds, edges, chunks, edge_type_emb, W_src, b_src, W_msg,
           W_intent, b_intent, W_ibn, b_ibn):
    del chunks
    n = entity_embeds.shape[0]
    e = edges.shape[0]
    ep = _EPAD - e
    src = jnp.pad(edges[:, 0], (0, ep))
    tgt = jnp.pad(edges[:, 1], (0, ep))
    tt = jnp.pad(edges[:, 2], (0, ep))
    et = jnp.pad(edges[:, 3], (0, ep))
    idxv3 = (et * _NP + src).reshape(_NW, _NCH, _C)
    idxu3 = ((tt * _NP + tgt) * _NET + et).reshape(_NW, _NCH, _C)
    tgt3 = tgt.reshape(_NW, _NCH, _C)
    idxa2 = idxv3.reshape(16, _NCH2, _C)
    tgt2 = tgt3.reshape(16, _NCH2, _C)

    x0 = jnp.pad(entity_embeds, ((0, _NP - n), (0, 0)))

    def layer(_, x):
        a, v, u = _tc_pre(x, W_msg, W_src, b_src, edge_type_emb,
                          W_intent, b_intent, W_ibn, b_ibn)
        es, den = _sc_scores(idxv3, tgt3, idxu3, v.reshape(-1),
                             u[:, :, :_NET].reshape(-1))
        return _sc_messages(idxa2, tgt2, es.reshape(16, _NCH2, _C),
                            den[0] + den[1], a.reshape(_NET * _NP, _D))

    x = lax.fori_loop(0, _LAYERS, layer, x0)
    return x[:n]
